# baseline (device time: 9345 ns/iter reference)
import jax
import jax.numpy as jnp
from jax import lax
from jax.experimental import pallas as pl
from jax.experimental.pallas import tpu as pltpu

N_DEV = 8
_ROUND_MASKS = (1, 3, 4)


def kernel(x):
    m_per, n = x.shape

    def body(x_hbm, out_ref, x_vmem, send_buf, recv_buf, copy_sem,
             send_sems, recv_sems, bsems):
        my_pos = lax.axis_index("i")
        barrier_sem = pltpu.get_barrier_semaphore()

        def round_sem(r):
            return barrier_sem if r == 0 else bsems.at[r - 1]

        def signal_round(r):
            pl.semaphore_signal(
                round_sem(r),
                inc=1,
                device_id=(jnp.bitwise_xor(my_pos, _ROUND_MASKS[r]),),
                device_id_type=pl.DeviceIdType.MESH,
            )

        signal_round(0)
        cp = pltpu.make_async_copy(x_hbm, x_vmem, copy_sem)
        cp.start()

        pl.semaphore_wait(round_sem(0), 1)
        signal_round(1)
        pl.semaphore_wait(round_sem(1), 1)
        signal_round(2)

        cp.wait()
        xv = x_vmem[:, :]
        val = jnp.max(xv, axis=0)
        row_ids = lax.broadcasted_iota(jnp.int32, (m_per, n), 0)
        masked = jnp.where(xv == val[None, :], row_ids, m_per * N_DEV)
        local_idx = jnp.min(masked, axis=0)
        gidx = (my_pos * m_per + local_idx).astype(jnp.float32)

        send_buf[0, :] = val
        send_buf[1, :] = gidx
        recv_buf[N_DEV - 1, 0, :] = val
        recv_buf[N_DEV - 1, 1, :] = gidx

        pl.semaphore_wait(round_sem(2), 1)

        rdmas = []
        for d in range(1, N_DEV):
            target = lax.rem(my_pos + d, N_DEV)
            rdma = pltpu.make_async_remote_copy(
                src_ref=send_buf,
                dst_ref=recv_buf.at[d - 1],
                send_sem=send_sems.at[d - 1],
                recv_sem=recv_sems.at[d - 1],
                device_id=(target,),
                device_id_type=pl.DeviceIdType.MESH,
            )
            rdma.start()
            rdmas.append(rdma)
        for rdma in rdmas:
            rdma.wait()

        vals = recv_buf[:, 0, :]
        idxs = recv_buf[:, 1, :]
        best_v = jnp.max(vals, axis=0)
        big = jnp.float32(m_per * N_DEV)
        best_i = jnp.min(jnp.where(vals == best_v[None, :], idxs, big), axis=0)

        out_ref[0, :] = best_v
        out_ref[1, :] = best_i

    return pl.pallas_call(
        body,
        out_shape=jax.ShapeDtypeStruct((2, n), jnp.float32),
        in_specs=[pl.BlockSpec(memory_space=pl.ANY)],
        out_specs=pl.BlockSpec(memory_space=pltpu.VMEM),
        scratch_shapes=[
            pltpu.VMEM((m_per, n), jnp.float32),
            pltpu.VMEM((2, n), jnp.float32),
            pltpu.VMEM((N_DEV, 2, n), jnp.float32),
            pltpu.SemaphoreType.DMA,
            pltpu.SemaphoreType.DMA((N_DEV - 1,)),
            pltpu.SemaphoreType.DMA((N_DEV - 1,)),
            pltpu.SemaphoreType.REGULAR((2,)),
        ],
        compiler_params=pltpu.CompilerParams(collective_id=0),
    )(x)


# device time: 9178 ns/iter; 1.0182x vs baseline; 1.0182x over previous
import jax
import jax.numpy as jnp
from jax import lax
from jax.experimental import pallas as pl
from jax.experimental.pallas import tpu as pltpu

N_DEV = 8
_ROUND_MASKS = (1, 3, 4)
_SENDS_AFTER_ROUND = (
    (1,),
    (3, 2),
    (6, 5, 7, 4),
)


def kernel(x):
    m_per, n = x.shape

    def body(x_ref, out_ref, send_buf, recv_buf, send_sems, recv_sems, bsems):
        my_pos = lax.axis_index("i")
        barrier_sem = pltpu.get_barrier_semaphore()

        def round_sem(r):
            return barrier_sem if r == 0 else bsems.at[r - 1]

        def signal_round(r):
            pl.semaphore_signal(
                round_sem(r),
                inc=1,
                device_id=(jnp.bitwise_xor(my_pos, _ROUND_MASKS[r]),),
                device_id_type=pl.DeviceIdType.MESH,
            )

        def send_to(mask):
            rdma = pltpu.make_async_remote_copy(
                src_ref=send_buf,
                dst_ref=recv_buf.at[mask - 1],
                send_sem=send_sems.at[mask - 1],
                recv_sem=recv_sems.at[mask - 1],
                device_id=(jnp.bitwise_xor(my_pos, mask),),
                device_id_type=pl.DeviceIdType.MESH,
            )
            rdma.start()
            return rdma

        signal_round(0)

        xv = x_ref[:, :]
        val = jnp.max(xv, axis=0)
        row_ids = lax.broadcasted_iota(jnp.int32, (m_per, n), 0)
        masked = jnp.where(xv == val[None, :], row_ids, m_per * N_DEV)
        local_idx = jnp.min(masked, axis=0)
        gidx = (my_pos * m_per + local_idx).astype(jnp.float32)

        send_buf[0, :] = val
        send_buf[1, :] = gidx
        recv_buf[N_DEV - 1, 0, :] = val
        recv_buf[N_DEV - 1, 1, :] = gidx

        rdmas = []
        for r in range(3):
            pl.semaphore_wait(round_sem(r), 1)
            if r < 2:
                signal_round(r + 1)
            rdmas.extend(send_to(m) for m in _SENDS_AFTER_ROUND[r])
        for rdma in rdmas:
            rdma.wait()

        vals = recv_buf[:, 0, :]
        idxs = recv_buf[:, 1, :]
        best_v = jnp.max(vals, axis=0)
        big = jnp.float32(m_per * N_DEV)
        best_i = jnp.min(jnp.where(vals == best_v[None, :], idxs, big), axis=0)

        out_ref[0, :] = best_v
        out_ref[1, :] = best_i

    return pl.pallas_call(
        body,
        out_shape=jax.ShapeDtypeStruct((2, n), jnp.float32),
        in_specs=[pl.BlockSpec(memory_space=pltpu.VMEM)],
        out_specs=pl.BlockSpec(memory_space=pltpu.VMEM),
        scratch_shapes=[
            pltpu.VMEM((2, n), jnp.float32),
            pltpu.VMEM((N_DEV, 2, n), jnp.float32),
            pltpu.SemaphoreType.DMA((N_DEV - 1,)),
            pltpu.SemaphoreType.DMA((N_DEV - 1,)),
            pltpu.SemaphoreType.REGULAR((2,)),
        ],
        compiler_params=pltpu.CompilerParams(collective_id=0),
    )(x)


# device time: 8678 ns/iter; 1.0769x vs baseline; 1.0576x over previous
import jax
import jax.numpy as jnp
from jax import lax
from jax.experimental import pallas as pl
from jax.experimental.pallas import tpu as pltpu

N_DEV = 8
_PEER_MASKS = (6, 2, 5, 7, 1, 3, 4)


def kernel(x):
    m_per, n = x.shape

    def body(x_ref, out_ref, send_buf, recv_buf, send_sems, recv_sems):
        my_pos = lax.axis_index("i")

        barrier_sem = pltpu.get_barrier_semaphore()
        for mask in _PEER_MASKS:
            pl.semaphore_signal(
                barrier_sem,
                inc=1,
                device_id=(jnp.bitwise_xor(my_pos, mask),),
                device_id_type=pl.DeviceIdType.MESH,
            )

        xv = x_ref[:, :]
        val = jnp.max(xv, axis=0)
        row_ids = lax.broadcasted_iota(jnp.int32, (m_per, n), 0)
        masked = jnp.where(xv == val[None, :], row_ids, m_per * N_DEV)
        local_idx = jnp.min(masked, axis=0)
        gidx = (my_pos * m_per + local_idx).astype(jnp.float32)

        send_buf[0, :] = val
        send_buf[1, :] = gidx
        recv_buf[N_DEV - 1, 0, :] = val
        recv_buf[N_DEV - 1, 1, :] = gidx

        pl.semaphore_wait(barrier_sem, N_DEV - 1)

        rdmas = []
        for mask in _PEER_MASKS:
            rdma = pltpu.make_async_remote_copy(
                src_ref=send_buf,
                dst_ref=recv_buf.at[mask - 1],
                send_sem=send_sems.at[mask - 1],
                recv_sem=recv_sems.at[mask - 1],
                device_id=(jnp.bitwise_xor(my_pos, mask),),
                device_id_type=pl.DeviceIdType.MESH,
            )
            rdma.start()
            rdmas.append(rdma)
        for rdma in rdmas:
            rdma.wait()

        vals = recv_buf[:, 0, :]
        idxs = recv_buf[:, 1, :]
        best_v = jnp.max(vals, axis=0)
        big = jnp.float32(m_per * N_DEV)
        best_i = jnp.min(jnp.where(vals == best_v[None, :], idxs, big), axis=0)

        out_ref[0, :] = best_v
        out_ref[1, :] = best_i

    return pl.pallas_call(
        body,
        out_shape=jax.ShapeDtypeStruct((2, n), jnp.float32),
        in_specs=[pl.BlockSpec(memory_space=pltpu.VMEM)],
        out_specs=pl.BlockSpec(memory_space=pltpu.VMEM),
        scratch_shapes=[
            pltpu.VMEM((2, n), jnp.float32),
            pltpu.VMEM((N_DEV, 2, n), jnp.float32),
            pltpu.SemaphoreType.DMA((N_DEV - 1,)),
            pltpu.SemaphoreType.DMA((N_DEV - 1,)),
        ],
        compiler_params=pltpu.CompilerParams(collective_id=0),
    )(x)
